# baseline (speedup 1.0000x reference)
import functools
import math

import jax
import jax.numpy as jnp
from jax import lax
from jax.experimental import pallas as pl
from jax.experimental.pallas import tpu as pltpu
from jax.experimental.pallas import tpu_sc as plsc

N_MATCHES = 16384
NC = 2
NS = 16
NW = NC * NS
B_PER_W = N_MATCHES // NW
CHUNK = 64
N_CHUNKS = B_PER_W // CHUNK
LANES = 16
_K = math.log(10.0) / 400.0

_mesh = plsc.VectorSubcoreMesh(core_axis_name="c", subcore_axis_name="s")


@functools.partial(
    pl.kernel,
    out_type=jax.ShapeDtypeStruct((N_MATCHES,), jnp.float32),
    mesh=_mesh,
    scratch_types=[
        pltpu.VMEM((N_CHUNKS, CHUNK), jnp.int32),
        pltpu.VMEM((N_CHUNKS, CHUNK), jnp.int32),
        pltpu.VMEM((B_PER_W,), jnp.float32),
        pltpu.VMEM((B_PER_W,), jnp.float32),
        pltpu.VMEM((B_PER_W,), jnp.float32),
        pltpu.SemaphoreType.DMA,
        pltpu.SemaphoreType.DMA,
    ],
)
def _elo_sc(home_hbm, away_hbm, rating_hbm, out_hbm,
            hidx, aidx, hval, aval, res, sem, sem_i):
    wid = lax.axis_index("s") * NC + lax.axis_index("c")
    ci_h = pltpu.async_copy(home_hbm.at[wid], hidx, sem_i)
    ci_a = pltpu.async_copy(away_hbm.at[wid], aidx, sem_i)

    copies = []
    ci_h.wait()
    for c in range(N_CHUNKS):
        copies.append(pltpu.async_copy(
            rating_hbm.at[hidx.at[c]], hval.at[pl.ds(c * CHUNK, CHUNK)], sem))
    ci_a.wait()
    for c in range(N_CHUNKS):
        copies.append(pltpu.async_copy(
            rating_hbm.at[aidx.at[c]], aval.at[pl.ds(c * CHUNK, CHUNK)], sem))
    for cp in copies:
        cp.wait()

    def body(i, _):
        h = hval[pl.ds(i * LANES, LANES)]
        a = aval[pl.ds(i * LANES, LANES)]
        res[pl.ds(i * LANES, LANES)] = 1.0 / (1.0 + jnp.exp((a - h) * _K))
        return 0

    lax.fori_loop(0, B_PER_W // LANES, body, 0)
    pltpu.sync_copy(res, out_hbm.at[pl.ds(wid * B_PER_W, B_PER_W)])


def kernel(matches, rating):
    home = matches[0].reshape(NW, N_CHUNKS, CHUNK)
    away = matches[1].reshape(NW, N_CHUNKS, CHUNK)
    return _elo_sc(home, away, rating)


# speedup vs baseline: 1.0164x; 1.0164x over previous
import functools
import math

import jax
import jax.numpy as jnp
from jax import lax
from jax.experimental import pallas as pl
from jax.experimental.pallas import tpu as pltpu
from jax.experimental.pallas import tpu_sc as plsc

N_MATCHES = 16384
NC = 2
NS = 16
NW = NC * NS
B_PER_W = N_MATCHES // NW
CHUNK = 128
N_CHUNKS = B_PER_W // CHUNK
LANES = 16
_K = math.log(10.0) / 400.0

_mesh = plsc.VectorSubcoreMesh(core_axis_name="c", subcore_axis_name="s")


@functools.partial(
    pl.kernel,
    out_type=jax.ShapeDtypeStruct((N_MATCHES,), jnp.float32),
    mesh=_mesh,
    scratch_types=[
        pltpu.VMEM((N_CHUNKS, CHUNK), jnp.int32),
        pltpu.VMEM((N_CHUNKS, CHUNK), jnp.int32),
        pltpu.VMEM((B_PER_W,), jnp.float32),
        pltpu.VMEM((B_PER_W,), jnp.float32),
        pltpu.VMEM((B_PER_W,), jnp.float32),
        pltpu.SemaphoreType.DMA,
        pltpu.SemaphoreType.DMA,
    ],
)
def _elo_sc(home_hbm, away_hbm, rating_hbm, out_hbm,
            hidx, aidx, hval, aval, res, sem, sem_i):
    wid = lax.axis_index("s") * NC + lax.axis_index("c")
    ci_h = pltpu.async_copy(home_hbm.at[wid], hidx, sem_i)
    ci_a = pltpu.async_copy(away_hbm.at[wid], aidx, sem_i)

    copies = []
    ci_h.wait()
    for c in range(N_CHUNKS):
        copies.append(pltpu.async_copy(
            rating_hbm.at[hidx.at[c]], hval.at[pl.ds(c * CHUNK, CHUNK)], sem))
    ci_a.wait()
    for c in range(N_CHUNKS):
        copies.append(pltpu.async_copy(
            rating_hbm.at[aidx.at[c]], aval.at[pl.ds(c * CHUNK, CHUNK)], sem))
    for cp in copies:
        cp.wait()

    @plsc.parallel_loop(0, B_PER_W, LANES, unroll=4)
    def body(i):
        h = hval[pl.ds(i, LANES)]
        a = aval[pl.ds(i, LANES)]
        res[pl.ds(i, LANES)] = 1.0 / (1.0 + jnp.exp((a - h) * _K))
    pltpu.sync_copy(res, out_hbm.at[pl.ds(wid * B_PER_W, B_PER_W)])


def kernel(matches, rating):
    home = matches[0].reshape(NW, N_CHUNKS, CHUNK)
    away = matches[1].reshape(NW, N_CHUNKS, CHUNK)
    return _elo_sc(home, away, rating)
